# baseline (speedup 1.0000x reference)
import functools

import jax
import jax.numpy as jnp
from jax import lax
from jax.experimental import pallas as pl
from jax.experimental.pallas import tpu as pltpu
from jax.experimental.pallas import tpu_sc as plsc

_WD = 0.01
_B = 16384
_D = 16
_NC = 2
_NS = 16
_NW = _NC * _NS
_BPW = _B // _NW
_CHUNK = 128
_NCHUNK = _BPW // _CHUNK


def _sc_body(w_hbm, h_hbm, u_hbm, i_hbm, j_hbm, out_hbm,
             u_v, i_v, j_v, ue_v, ie_v, je_v, res_v, sem):
    wid = lax.axis_index("s") * _NC + lax.axis_index("c")
    base = wid * _BPW

    pltpu.sync_copy(u_hbm.at[pl.ds(base, _BPW)], u_v)
    pltpu.sync_copy(i_hbm.at[pl.ds(base, _BPW)], i_v)
    pltpu.sync_copy(j_hbm.at[pl.ds(base, _BPW)], j_v)

    copies = []
    for k in range(_NCHUNK):
        sl = pl.ds(k * _CHUNK, _CHUNK)
        copies.append(pltpu.async_copy(w_hbm.at[u_v.at[sl]], ue_v.at[sl], sem))
        copies.append(pltpu.async_copy(h_hbm.at[i_v.at[sl]], ie_v.at[sl], sem))
        copies.append(pltpu.async_copy(h_hbm.at[j_v.at[sl]], je_v.at[sl], sem))
    for c in copies:
        c.wait()

    iota16 = lax.iota(jnp.int32, 16)
    cols = [jnp.full((16,), d, jnp.int32) for d in range(_D)]

    def block(t, carry):
        ls_acc, reg_acc = carry
        rows = t * 16 + iota16
        x = jnp.zeros((16,), jnp.float32)
        reg = reg_acc
        for d in range(_D):
            cu = plsc.load_gather(ue_v, [rows, cols[d]])
            ci = plsc.load_gather(ie_v, [rows, cols[d]])
            cj = plsc.load_gather(je_v, [rows, cols[d]])
            x = x + cu * (ci - cj)
            reg = reg + cu * cu + ci * ci + cj * cj
        w = jnp.exp(-jnp.abs(x))
        t_ = w / (w + 2.0)
        t2 = t_ * t_
        poly = 1.0 + t2 * (1.0 / 3.0 + t2 * (1.0 / 5.0 + t2 * (
            1.0 / 7.0 + t2 * (1.0 / 9.0 + t2 * (1.0 / 11.0)))))
        ls = jnp.minimum(x, 0.0) - 2.0 * t_ * poly
        return (ls_acc + ls, reg)

    zero = jnp.zeros((16,), jnp.float32)
    ls_acc, reg_acc = lax.fori_loop(0, _BPW // 16, block, (zero, zero))

    res_v[...] = _WD * reg_acc - ls_acc
    pltpu.sync_copy(res_v, out_hbm.at[wid])


@functools.partial(jax.jit, static_argnames=())
def _bpr_partials(w, h, u, i, j):
    mesh = plsc.VectorSubcoreMesh(core_axis_name="c", subcore_axis_name="s")
    return pl.kernel(
        _sc_body,
        out_type=jax.ShapeDtypeStruct((_NW, 16), jnp.float32),
        mesh=mesh,
        compiler_params=pltpu.CompilerParams(
            needs_layout_passes=False, use_tc_tiling_on_sc=False),
        scratch_types=[
            pltpu.VMEM((_BPW,), jnp.int32),
            pltpu.VMEM((_BPW,), jnp.int32),
            pltpu.VMEM((_BPW,), jnp.int32),
            pltpu.VMEM((_BPW, _D), jnp.float32),
            pltpu.VMEM((_BPW, _D), jnp.float32),
            pltpu.VMEM((_BPW, _D), jnp.float32),
            pltpu.VMEM((16,), jnp.float32),
            pltpu.SemaphoreType.DMA,
        ],
    )(w, h, u, i, j)


def kernel(W, H, u, i, i_pop, j, j_pop):
    del i_pop, j_pop
    partials = _bpr_partials(
        W, H, u.astype(jnp.int32), i.astype(jnp.int32), j.astype(jnp.int32))
    return jnp.sum(partials)
